# Initial kernel scaffold; baseline (speedup 1.0000x reference)
import jax
import jax.numpy as jnp
from jax.experimental import pallas as pl


def kernel(x, edge_index, batch, target, params):
    raise NotImplementedError("write your pallas kernel here")


# speedup vs baseline: 4.4850x; 4.4850x over previous
import jax
import jax.numpy as jnp
from jax import lax
from jax.experimental import pallas as pl
from jax.experimental.pallas import tpu as pltpu
from jax.experimental.pallas import tpu_sc as plsc

NNODES = 50000
NGRAPH = 256
FEAT = 32
NPAD = 50048
NSUB = 16
NCORE = 2
NTILE = NCORE * NSUB
CHUNK = 128
SB = 40
HF = 40
CHUNK0 = 64
SB0 = 8
BN = 2000
BEPS = 1e-5
HIGH = lax.Precision.HIGHEST


def _sc_agg_body(y_hbm, src_hbm, dst_hbm, zeros_hbm, out_hbm,
                 src_v, dst_v, rows_v, acc_sh, sem):
    c = lax.axis_index("c")
    s = lax.axis_index("s")
    wid = c * NSUB + s
    cpt = src_hbm.shape[0] // NTILE
    rpt = NPAD // NSUB

    pltpu.sync_copy(zeros_hbm.at[pl.ds(s * rpt, rpt)],
                    acc_sh.at[pl.ds(s * rpt, rpt)])
    plsc.subcore_barrier()

    def blk_body(b, carry):
        base = wid * cpt + b * SB
        pltpu.sync_copy(src_hbm.at[pl.ds(base, SB)], src_v)
        pltpu.sync_copy(dst_hbm.at[pl.ds(base, SB)], dst_v)

        def chunk_body(j, carry2):
            pltpu.async_copy(y_hbm.at[src_v.at[j]], rows_v, sem).wait()
            pltpu.sync_copy(rows_v, acc_sh.at[dst_v.at[j]], add=True)
            return carry2
        lax.fori_loop(0, SB, chunk_body, 0)
        return carry
    lax.fori_loop(0, cpt // SB, blk_body, 0)

    plsc.subcore_barrier()
    pltpu.sync_copy(acc_sh.at[pl.ds(s * rpt, rpt)],
                    out_hbm.at[c, pl.ds(s * rpt, rpt)])


def _sc_aggregate(y, src2d, dst2d, zeros_pad):
    mesh = plsc.VectorSubcoreMesh(core_axis_name="c", subcore_axis_name="s")
    return pl.kernel(
        _sc_agg_body,
        out_type=jax.ShapeDtypeStruct((NCORE, NPAD, FEAT), jnp.float32),
        mesh=mesh,
        scratch_types=[
            pltpu.VMEM((SB, CHUNK), jnp.int32),
            pltpu.VMEM((SB, CHUNK), jnp.int32),
            pltpu.VMEM((CHUNK, FEAT), jnp.float32),
            pltpu.VMEM_SHARED((NPAD, FEAT), jnp.float32),
            pltpu.SemaphoreType.DMA,
        ],
        compiler_params=pltpu.CompilerParams(use_tc_tiling_on_sc=False),
    )(y, src2d, dst2d, zeros_pad)


def _sc_agg0_body(x_hbm, src_hbm, dst_hbm, zeros_hbm, out_hbm,
                  src_v, dst_v, rows_v, acc_sh, sem):
    c = lax.axis_index("c")
    s = lax.axis_index("s")
    cpt = src_hbm.shape[0] // NSUB
    rpt = NPAD // NSUB

    pltpu.sync_copy(zeros_hbm.at[pl.ds(s * rpt, rpt)],
                    acc_sh.at[pl.ds(s * rpt, rpt)])
    plsc.subcore_barrier()

    def blk_body(b, carry):
        base = s * cpt + b * SB0
        pltpu.sync_copy(src_hbm.at[pl.ds(base, SB0)], src_v)
        pltpu.sync_copy(dst_hbm.at[pl.ds(base, SB0)], dst_v)

        def chunk_body(j, carry2):
            pltpu.async_copy(x_hbm.at[c].at[src_v.at[j]], rows_v, sem).wait()
            pltpu.sync_copy(rows_v, acc_sh.at[dst_v.at[j]], add=True)
            return carry2
        lax.fori_loop(0, SB0, chunk_body, 0)
        return carry
    lax.fori_loop(0, cpt // SB0, blk_body, 0)

    plsc.subcore_barrier()
    pltpu.sync_copy(acc_sh.at[pl.ds(s * rpt, rpt)],
                    out_hbm.at[c, pl.ds(s * rpt, rpt)])


def _sc_aggregate0(x3d, src2d, dst2d, zeros_half):
    mesh = plsc.VectorSubcoreMesh(core_axis_name="c", subcore_axis_name="s")
    return pl.kernel(
        _sc_agg0_body,
        out_type=jax.ShapeDtypeStruct((NCORE, NPAD, HF), jnp.float32),
        mesh=mesh,
        scratch_types=[
            pltpu.VMEM((SB0, CHUNK0), jnp.int32),
            pltpu.VMEM((SB0, CHUNK0), jnp.int32),
            pltpu.VMEM((CHUNK0, HF), jnp.float32),
            pltpu.VMEM_SHARED((NPAD, HF), jnp.float32),
            pltpu.SemaphoreType.DMA,
        ],
        compiler_params=pltpu.CompilerParams(use_tc_tiling_on_sc=False),
    )(x3d, src2d, dst2d, zeros_half)


def _stats_update(st_ref, z):
    ssum = jnp.sum(z, axis=0)[None, :]
    ssq = jnp.sum(z * z, axis=0)[None, :]
    blk = jnp.concatenate([ssum, ssq, jnp.zeros((6, FEAT), jnp.float32)],
                          axis=0)

    @pl.when(pl.program_id(0) == 0)
    def _init():
        st_ref[...] = blk

    @pl.when(pl.program_id(0) > 0)
    def _acc():
        st_ref[...] += blk


def _gin0_body(xa_ref, xb_ref, p0_ref, p1_ref, wa_ref, wb_ref, b1_ref,
               w2_ref, b2_ref, z_ref, st_ref):
    z0a = xa_ref[...] + p0_ref[...]
    z0b = xb_ref[...] + p1_ref[...]
    u = (jnp.dot(z0a, wa_ref[...], preferred_element_type=jnp.float32)
         + jnp.dot(z0b, wb_ref[...], preferred_element_type=jnp.float32)
         + b1_ref[...])
    a = jnp.maximum(u, 0.0)
    v = jnp.dot(a, w2_ref[...],
                preferred_element_type=jnp.float32) + b2_ref[...]
    z = jnp.maximum(v, 0.0)
    z_ref[...] = z
    _stats_update(st_ref, z)


def _gin0(xa, xb, p0, p1, wa, wb, b1row, w2, b2row):
    n = xa.shape[0]
    rb = lambda i: (i, 0)
    cb = lambda i: (0, 0)
    return pl.pallas_call(
        _gin0_body,
        grid=(n // BN,),
        in_specs=[
            pl.BlockSpec((BN, HF), rb),
            pl.BlockSpec((BN, HF), rb),
            pl.BlockSpec((BN, HF), rb),
            pl.BlockSpec((BN, HF), rb),
            pl.BlockSpec((HF, FEAT), cb),
            pl.BlockSpec((HF, FEAT), cb),
            pl.BlockSpec((1, FEAT), cb),
            pl.BlockSpec((FEAT, FEAT), cb),
            pl.BlockSpec((1, FEAT), cb),
        ],
        out_specs=[
            pl.BlockSpec((BN, FEAT), rb),
            pl.BlockSpec((8, FEAT), cb),
        ],
        out_shape=[
            jax.ShapeDtypeStruct((n, FEAT), jnp.float32),
            jax.ShapeDtypeStruct((8, FEAT), jnp.float32),
        ],
    )(xa, xb, p0, p1, wa, wb, b1row, w2, b2row)


def _gin_body(z_ref, p0_ref, p1_ref, deg_ref, s_ref, t_ref, w1_ref, b1_ref,
              w2_ref, b2_ref, zo_ref, st_ref):
    z0 = ((z_ref[...] + p0_ref[...] + p1_ref[...]) * s_ref[...]
          + (1.0 + deg_ref[...]) * t_ref[...])
    u = jnp.dot(z0, w1_ref[...],
                preferred_element_type=jnp.float32) + b1_ref[...]
    a = jnp.maximum(u, 0.0)
    v = jnp.dot(a, w2_ref[...],
                preferred_element_type=jnp.float32) + b2_ref[...]
    z = jnp.maximum(v, 0.0)
    zo_ref[...] = z
    _stats_update(st_ref, z)


def _gin(z, p0, p1, deg, srow, trow, w1, b1row, w2, b2row):
    n = z.shape[0]
    rb = lambda i: (i, 0)
    cb = lambda i: (0, 0)
    return pl.pallas_call(
        _gin_body,
        grid=(n // BN,),
        in_specs=[
            pl.BlockSpec((BN, FEAT), rb),
            pl.BlockSpec((BN, FEAT), rb),
            pl.BlockSpec((BN, FEAT), rb),
            pl.BlockSpec((BN, 1), rb),
            pl.BlockSpec((1, FEAT), cb),
            pl.BlockSpec((1, FEAT), cb),
            pl.BlockSpec((FEAT, FEAT), cb),
            pl.BlockSpec((1, FEAT), cb),
            pl.BlockSpec((FEAT, FEAT), cb),
            pl.BlockSpec((1, FEAT), cb),
        ],
        out_specs=[
            pl.BlockSpec((BN, FEAT), rb),
            pl.BlockSpec((8, FEAT), cb),
        ],
        out_shape=[
            jax.ShapeDtypeStruct((n, FEAT), jnp.float32),
            jax.ShapeDtypeStruct((8, FEAT), jnp.float32),
        ],
    )(z, p0, p1, deg, srow, trow, w1, b1row, w2, b2row)


def _pool_body(z_ref, b_ref, o_ref):
    z = z_ref[...]
    brow = b_ref[...].reshape(1, BN)
    m = (lax.broadcasted_iota(jnp.int32, (NGRAPH, BN), 0) == brow
         ).astype(jnp.float32)
    zz = jnp.concatenate([z, jnp.ones((BN, FEAT), jnp.float32)], axis=1)
    blk = jnp.dot(m, zz, preferred_element_type=jnp.float32, precision=HIGH)

    @pl.when(pl.program_id(0) == 0)
    def _init():
        o_ref[...] = blk

    @pl.when(pl.program_id(0) > 0)
    def _acc():
        o_ref[...] += blk


def _pool(z, batch3d):
    n = z.shape[0]
    grid = n // BN
    return pl.pallas_call(
        _pool_body,
        grid=(grid,),
        in_specs=[
            pl.BlockSpec((BN, FEAT), lambda i: (i, 0)),
            pl.BlockSpec((1, 1, BN), lambda i: (i, 0, 0)),
        ],
        out_specs=pl.BlockSpec((NGRAPH, 2 * FEAT), lambda i: (0, 0)),
        out_shape=jax.ShapeDtypeStruct((NGRAPH, 2 * FEAT), jnp.float32),
    )(z, batch3d)


BH = 16


def _head_body(pool_ref, st_ref, w0_ref, b0_ref, tgt_ref, emb_ref, wf_ref,
               wbig_ref, bxt_ref, w1_ref, b1_ref, w2_ref, b2_ref,
               wo_ref, bo_ref, o_ref):
    poolb = pool_ref[...]
    seg = poolb[:, :FEAT]
    cnt = poolb[:, FEAT:]
    s4 = st_ref[0:1, :]
    t4 = st_ref[1:2, :]
    g = (seg * s4 + cnt * t4) / jnp.maximum(cnt, 1.0)
    xd = jnp.maximum(jnp.dot(g, w0_ref[...], preferred_element_type=jnp.float32,
                             precision=HIGH) + b0_ref[...], 0.0)

    tgt = tgt_ref[...]
    emb = emb_ref[...]
    wf = wf_ref[...]
    rows = []
    for r in range(BH):
        tr = tgt[r:r + 1, :]
        ohT = (lax.broadcasted_iota(jnp.int32, (32, 1000), 0) == tr
               ).astype(jnp.float32)
        q = jnp.dot(ohT, wf, preferred_element_type=jnp.float32,
                    precision=HIGH)
        u = lax.dot_general(emb, q, (((0,), (0,)), ((), ())),
                            preferred_element_type=jnp.float32,
                            precision=HIGH)
        rows.append(u.reshape(1, 128 * 256))
    uflat = jnp.concatenate(rows, axis=0)
    xt = jnp.dot(uflat, wbig_ref[...], preferred_element_type=jnp.float32,
                 precision=HIGH) + bxt_ref[...]
    xc = jnp.concatenate([xd, xt], axis=1)
    h1 = jnp.maximum(jnp.dot(xc, w1_ref[...], preferred_element_type=jnp.float32,
                             precision=HIGH) + b1_ref[...], 0.0)
    h2 = jnp.maximum(jnp.dot(h1, w2_ref[...], preferred_element_type=jnp.float32,
                             precision=HIGH) + b2_ref[...], 0.0)
    o_ref[...] = jnp.dot(h2, wo_ref[...], preferred_element_type=jnp.float32,
                         precision=HIGH) + bo_ref[...]


def _head(pool, st4, w0, b0, tgt, embp, wf, wbig, bxt, w1, b1, w2, b2, wo, bo):
    grid = NGRAPH // BH
    cm = lambda i: (0, 0)
    return pl.pallas_call(
        _head_body,
        grid=(grid,),
        in_specs=[
            pl.BlockSpec((BH, 2 * FEAT), lambda i: (i, 0)),
            pl.BlockSpec((8, FEAT), cm),
            pl.BlockSpec((FEAT, 128), cm),
            pl.BlockSpec((1, 128), cm),
            pl.BlockSpec((BH, 1000), lambda i: (i, 0)),
            pl.BlockSpec((32, 128), cm),
            pl.BlockSpec((1000, 256), cm),
            pl.BlockSpec((128 * 256, 128), cm),
            pl.BlockSpec((1, 128), cm),
            pl.BlockSpec((256, 1024), cm),
            pl.BlockSpec((1, 1024), cm),
            pl.BlockSpec((1024, 256), cm),
            pl.BlockSpec((1, 256), cm),
            pl.BlockSpec((256, 128), cm),
            pl.BlockSpec((1, 128), cm),
        ],
        out_specs=pl.BlockSpec((BH, 128), lambda i: (i, 0)),
        out_shape=jax.ShapeDtypeStruct((NGRAPH, 128), jnp.float32),
    )(pool, st4, w0, b0, tgt, embp, wf, wbig, bxt, w1, b1, w2, b2, wo, bo)


def kernel(x, edge_index, batch, target, params):
    n = x.shape[0]
    e = edge_index.shape[1]
    src = edge_index[0]
    dst = edge_index[1]

    cpt = -(-e // (NTILE * CHUNK))
    cpt = -(-cpt // SB) * SB
    epad = cpt * NTILE * CHUNK
    src2d = jnp.concatenate(
        [src, jnp.zeros((epad - e,), jnp.int32)]).reshape(-1, CHUNK)
    dst2d = jnp.concatenate(
        [dst, jnp.full((epad - e,), n, jnp.int32)]).reshape(-1, CHUNK)
    zeros_pad = jnp.zeros((NPAD, FEAT), jnp.float32)

    cpt0 = -(-e // (NSUB * CHUNK0))
    cpt0 = -(-cpt0 // SB0) * SB0
    epad0 = cpt0 * NSUB * CHUNK0
    src2d0 = jnp.concatenate(
        [src, jnp.full((epad0 - e,), n, jnp.int32)]).reshape(-1, CHUNK0)
    dst2d0 = jnp.concatenate(
        [dst, jnp.zeros((epad0 - e,), jnp.int32)]).reshape(-1, CHUNK0)
    zeros_half = jnp.zeros((NPAD, HF), jnp.float32)

    x80 = jnp.concatenate(
        [x, jnp.ones((n, 1), jnp.float32), jnp.zeros((n, 1), jnp.float32)],
        axis=1)
    x80p = jnp.concatenate(
        [x80, jnp.zeros((NPAD - n, 2 * HF), jnp.float32)], axis=0)
    x3d = jnp.stack([x80p[:, :HF], x80p[:, HF:]])
    agg0 = _sc_aggregate0(x3d, src2d0, dst2d0, zeros_half)
    p0_0 = agg0[0, :n]
    p1_0 = agg0[1, :n]
    deg = p1_0[:, 38:39]

    w1_0 = params['gin0_l1']['W']
    wa = w1_0[:HF]
    wb = jnp.concatenate([w1_0[HF:], jnp.zeros((2, FEAT), jnp.float32)])
    z, stats = _gin0(x80p[:n, :HF], x80p[:n, HF:], p0_0, p1_0, wa, wb,
                     params['gin0_l1']['b'][None, :],
                     params['gin0_l2']['W'], params['gin0_l2']['b'][None, :])
    mu = stats[0] / n
    var = stats[1] / n - mu * mu
    s_bn = params['bn0_g'] / jnp.sqrt(var + BEPS)
    t_bn = params['bn0_b'] - mu * s_bn

    for i in range(1, 5):
        l1 = params['gin%d_l1' % i]
        l2 = params['gin%d_l2' % i]
        p = _sc_aggregate(z, src2d, dst2d, zeros_pad)
        z, stats = _gin(z, p[0, :n], p[1, :n], deg,
                        s_bn[None, :], t_bn[None, :],
                        l1['W'], l1['b'][None, :],
                        l2['W'], l2['b'][None, :])
        mu = stats[0] / n
        var = stats[1] / n - mu * mu
        s_bn = params['bn%d_g' % i] / jnp.sqrt(var + BEPS)
        t_bn = params['bn%d_b' % i] - mu * s_bn

    batch3d = batch.reshape(n // BN, 1, BN)
    pool = _pool(z, batch3d)

    st4 = jnp.concatenate([s_bn[None], t_bn[None],
                           jnp.zeros((6, FEAT), jnp.float32)], axis=0)

    embp = jnp.concatenate(
        [params['emb'], jnp.zeros((6, 128), jnp.float32)], axis=0)
    wf = params['conv_W'].transpose(1, 0, 2).reshape(1000, 256)
    bias_rep = jnp.repeat(params['conv_b'], 121)
    bxt = (params['fc1_xt']['b'] + bias_rep @ params['fc1_xt']['W'])[None, :]
    wxt3 = params['fc1_xt']['W'].reshape(32, 121, 128)
    pp = jnp.arange(128)[:, None] - jnp.arange(8)[None, :]
    valid = (pp >= 0) & (pp <= 120)
    gathered = wxt3[:, jnp.clip(pp, 0, 120), :]
    wbig = jnp.where(valid[None, :, :, None], gathered,
                     0.0).transpose(1, 0, 2, 3).reshape(128 * 256, 128)
    wo = jnp.concatenate(
        [params['out']['W'], jnp.zeros((256, 127), jnp.float32)], axis=1)
    bo = jnp.concatenate(
        [params['out']['b'], jnp.zeros((127,), jnp.float32)])[None, :]

    res = _head(pool, st4,
                params['fc1_xd']['W'], params['fc1_xd']['b'][None, :],
                target, embp, wf,
                wbig, bxt,
                params['fc1']['W'], params['fc1']['b'][None, :],
                params['fc2']['W'], params['fc2']['b'][None, :],
                wo, bo)
    return res[:, :1]


# speedup vs baseline: 4.5070x; 1.0049x over previous
import jax
import jax.numpy as jnp
from jax import lax
from jax.experimental import pallas as pl
from jax.experimental.pallas import tpu as pltpu
from jax.experimental.pallas import tpu_sc as plsc

NNODES = 50000
NGRAPH = 256
FEAT = 32
NPAD = 50048
NSUB = 16
NCORE = 2
NTILE = NCORE * NSUB
CHUNK = 128
SB = 40
HF = 40
CHUNK0 = 64
SB0 = 8
BN = 2000
BEPS = 1e-5
HIGH = lax.Precision.HIGHEST


def _sc_agg_body(y_hbm, src_hbm, dst_hbm, zeros_hbm, out_hbm,
                 src_v, dst_v, rows_v, acc_sh, sem):
    c = lax.axis_index("c")
    s = lax.axis_index("s")
    wid = c * NSUB + s
    cpt = src_hbm.shape[0] // NTILE
    rpt = NPAD // NSUB

    pltpu.sync_copy(zeros_hbm.at[pl.ds(s * rpt, rpt)],
                    acc_sh.at[pl.ds(s * rpt, rpt)])
    plsc.subcore_barrier()

    def blk_body(b, carry):
        base = wid * cpt + b * SB
        pltpu.sync_copy(src_hbm.at[pl.ds(base, SB)], src_v)
        pltpu.sync_copy(dst_hbm.at[pl.ds(base, SB)], dst_v)

        def chunk_body(j, carry2):
            pltpu.async_copy(y_hbm.at[src_v.at[j]], rows_v, sem).wait()
            pltpu.sync_copy(rows_v, acc_sh.at[dst_v.at[j]], add=True)
            return carry2
        lax.fori_loop(0, SB, chunk_body, 0)
        return carry
    lax.fori_loop(0, cpt // SB, blk_body, 0)

    plsc.subcore_barrier()
    pltpu.sync_copy(acc_sh.at[pl.ds(s * rpt, rpt)],
                    out_hbm.at[c, pl.ds(s * rpt, rpt)])


def _sc_aggregate(y, src2d, dst2d, zeros_pad):
    mesh = plsc.VectorSubcoreMesh(core_axis_name="c", subcore_axis_name="s")
    return pl.kernel(
        _sc_agg_body,
        out_type=jax.ShapeDtypeStruct((NCORE, NPAD, FEAT), jnp.float32),
        mesh=mesh,
        scratch_types=[
            pltpu.VMEM((SB, CHUNK), jnp.int32),
            pltpu.VMEM((SB, CHUNK), jnp.int32),
            pltpu.VMEM((CHUNK, FEAT), jnp.float32),
            pltpu.VMEM_SHARED((NPAD, FEAT), jnp.float32),
            pltpu.SemaphoreType.DMA,
        ],
        compiler_params=pltpu.CompilerParams(use_tc_tiling_on_sc=False),
    )(y, src2d, dst2d, zeros_pad)


def _sc_agg0_body(x_hbm, src_hbm, dst_hbm, zeros_hbm, out_hbm,
                  src_v, dst_v, rows_v, acc_sh, sem):
    c = lax.axis_index("c")
    s = lax.axis_index("s")
    cpt = src_hbm.shape[0] // NSUB
    rpt = NPAD // NSUB

    pltpu.sync_copy(zeros_hbm.at[pl.ds(s * rpt, rpt)],
                    acc_sh.at[pl.ds(s * rpt, rpt)])
    plsc.subcore_barrier()

    def blk_body(b, carry):
        base = s * cpt + b * SB0
        pltpu.sync_copy(src_hbm.at[pl.ds(base, SB0)], src_v)
        pltpu.sync_copy(dst_hbm.at[pl.ds(base, SB0)], dst_v)

        def chunk_body(j, carry2):
            pltpu.async_copy(x_hbm.at[c].at[src_v.at[j]], rows_v, sem).wait()
            pltpu.sync_copy(rows_v, acc_sh.at[dst_v.at[j]], add=True)
            return carry2
        lax.fori_loop(0, SB0, chunk_body, 0)
        return carry
    lax.fori_loop(0, cpt // SB0, blk_body, 0)

    plsc.subcore_barrier()
    pltpu.sync_copy(acc_sh.at[pl.ds(s * rpt, rpt)],
                    out_hbm.at[c, pl.ds(s * rpt, rpt)])


def _sc_aggregate0(x3d, src2d, dst2d, zeros_half):
    mesh = plsc.VectorSubcoreMesh(core_axis_name="c", subcore_axis_name="s")
    return pl.kernel(
        _sc_agg0_body,
        out_type=jax.ShapeDtypeStruct((NCORE, NPAD, HF), jnp.float32),
        mesh=mesh,
        scratch_types=[
            pltpu.VMEM((SB0, CHUNK0), jnp.int32),
            pltpu.VMEM((SB0, CHUNK0), jnp.int32),
            pltpu.VMEM((CHUNK0, HF), jnp.float32),
            pltpu.VMEM_SHARED((NPAD, HF), jnp.float32),
            pltpu.SemaphoreType.DMA,
        ],
        compiler_params=pltpu.CompilerParams(use_tc_tiling_on_sc=False),
    )(x3d, src2d, dst2d, zeros_half)


def _stats_update(st_ref, z):
    ssum = jnp.sum(z, axis=0)[None, :]
    ssq = jnp.sum(z * z, axis=0)[None, :]
    blk = jnp.concatenate([ssum, ssq, jnp.zeros((6, FEAT), jnp.float32)],
                          axis=0)

    @pl.when(pl.program_id(0) == 0)
    def _init():
        st_ref[...] = blk

    @pl.when(pl.program_id(0) > 0)
    def _acc():
        st_ref[...] += blk


def _gin0_body(xa_ref, xb_ref, p0_ref, p1_ref, wa_ref, wb_ref, b1_ref,
               w2_ref, b2_ref, z_ref, st_ref):
    z0a = xa_ref[...] + p0_ref[...]
    z0b = xb_ref[...] + p1_ref[...]
    u = (jnp.dot(z0a, wa_ref[...], preferred_element_type=jnp.float32)
         + jnp.dot(z0b, wb_ref[...], preferred_element_type=jnp.float32)
         + b1_ref[...])
    a = jnp.maximum(u, 0.0)
    v = jnp.dot(a, w2_ref[...],
                preferred_element_type=jnp.float32) + b2_ref[...]
    z = jnp.maximum(v, 0.0)
    z_ref[...] = z
    _stats_update(st_ref, z)


def _gin0(xa, xb, p0, p1, wa, wb, b1row, w2, b2row):
    n = xa.shape[0]
    rb = lambda i: (i, 0)
    cb = lambda i: (0, 0)
    return pl.pallas_call(
        _gin0_body,
        grid=(n // BN,),
        in_specs=[
            pl.BlockSpec((BN, HF), rb),
            pl.BlockSpec((BN, HF), rb),
            pl.BlockSpec((BN, HF), rb),
            pl.BlockSpec((BN, HF), rb),
            pl.BlockSpec((HF, FEAT), cb),
            pl.BlockSpec((HF, FEAT), cb),
            pl.BlockSpec((1, FEAT), cb),
            pl.BlockSpec((FEAT, FEAT), cb),
            pl.BlockSpec((1, FEAT), cb),
        ],
        out_specs=[
            pl.BlockSpec((BN, FEAT), rb),
            pl.BlockSpec((8, FEAT), cb),
        ],
        out_shape=[
            jax.ShapeDtypeStruct((n, FEAT), jnp.float32),
            jax.ShapeDtypeStruct((8, FEAT), jnp.float32),
        ],
    )(xa, xb, p0, p1, wa, wb, b1row, w2, b2row)


def _gin_body(z_ref, p0_ref, p1_ref, deg_ref, s_ref, t_ref, w1_ref, b1_ref,
              w2_ref, b2_ref, zo_ref, st_ref):
    z0 = ((z_ref[...] + p0_ref[...] + p1_ref[...]) * s_ref[...]
          + (1.0 + deg_ref[...]) * t_ref[...])
    u = jnp.dot(z0, w1_ref[...],
                preferred_element_type=jnp.float32) + b1_ref[...]
    a = jnp.maximum(u, 0.0)
    v = jnp.dot(a, w2_ref[...],
                preferred_element_type=jnp.float32) + b2_ref[...]
    z = jnp.maximum(v, 0.0)
    zo_ref[...] = z
    _stats_update(st_ref, z)


def _gin(z, p0, p1, deg, srow, trow, w1, b1row, w2, b2row):
    n = z.shape[0]
    rb = lambda i: (i, 0)
    cb = lambda i: (0, 0)
    return pl.pallas_call(
        _gin_body,
        grid=(n // BN,),
        in_specs=[
            pl.BlockSpec((BN, FEAT), rb),
            pl.BlockSpec((BN, FEAT), rb),
            pl.BlockSpec((BN, FEAT), rb),
            pl.BlockSpec((BN, 1), rb),
            pl.BlockSpec((1, FEAT), cb),
            pl.BlockSpec((1, FEAT), cb),
            pl.BlockSpec((FEAT, FEAT), cb),
            pl.BlockSpec((1, FEAT), cb),
            pl.BlockSpec((FEAT, FEAT), cb),
            pl.BlockSpec((1, FEAT), cb),
        ],
        out_specs=[
            pl.BlockSpec((BN, FEAT), rb),
            pl.BlockSpec((8, FEAT), cb),
        ],
        out_shape=[
            jax.ShapeDtypeStruct((n, FEAT), jnp.float32),
            jax.ShapeDtypeStruct((8, FEAT), jnp.float32),
        ],
    )(z, p0, p1, deg, srow, trow, w1, b1row, w2, b2row)


def _pool_body(z_ref, b_ref, o_ref):
    z = z_ref[...]
    brow = b_ref[...].reshape(1, BN)
    m = (lax.broadcasted_iota(jnp.int32, (NGRAPH, BN), 0) == brow
         ).astype(jnp.float32)
    zz = jnp.concatenate([z, jnp.ones((BN, FEAT), jnp.float32)], axis=1)
    blk = jnp.dot(m, zz, preferred_element_type=jnp.float32, precision=HIGH)

    @pl.when(pl.program_id(0) == 0)
    def _init():
        o_ref[...] = blk

    @pl.when(pl.program_id(0) > 0)
    def _acc():
        o_ref[...] += blk


def _pool(z, batch3d):
    n = z.shape[0]
    grid = n // BN
    return pl.pallas_call(
        _pool_body,
        grid=(grid,),
        in_specs=[
            pl.BlockSpec((BN, FEAT), lambda i: (i, 0)),
            pl.BlockSpec((1, 1, BN), lambda i: (i, 0, 0)),
        ],
        out_specs=pl.BlockSpec((NGRAPH, 2 * FEAT), lambda i: (0, 0)),
        out_shape=jax.ShapeDtypeStruct((NGRAPH, 2 * FEAT), jnp.float32),
    )(z, batch3d)


BH = 16


def _head_body(pool_ref, st_ref, w0_ref, b0_ref, tgt_ref, emb_ref, wf_ref,
               wbig_ref, bxt_ref, w1_ref, b1_ref, w2_ref, b2_ref,
               wo_ref, bo_ref, o_ref):
    poolb = pool_ref[...]
    seg = poolb[:, :FEAT]
    cnt = poolb[:, FEAT:]
    s4 = st_ref[0:1, :]
    t4 = st_ref[1:2, :]
    g = (seg * s4 + cnt * t4) / jnp.maximum(cnt, 1.0)
    xd = jnp.maximum(jnp.dot(g, w0_ref[...], preferred_element_type=jnp.float32
                             ) + b0_ref[...], 0.0)

    tgt = tgt_ref[...]
    emb = emb_ref[...]
    wf = wf_ref[...]
    rows = []
    for r in range(BH):
        tr = tgt[r:r + 1, :]
        ohT = (lax.broadcasted_iota(jnp.int32, (32, 1000), 0) == tr
               ).astype(jnp.float32)
        q = jnp.dot(ohT, wf, preferred_element_type=jnp.float32,
                    precision=HIGH)
        u = lax.dot_general(emb, q, (((0,), (0,)), ((), ())),
                            preferred_element_type=jnp.float32,
                            precision=HIGH)
        rows.append(u.reshape(1, 128 * 256))
    uflat = jnp.concatenate(rows, axis=0)
    xt = jnp.dot(uflat, wbig_ref[...], preferred_element_type=jnp.float32,
                 precision=HIGH) + bxt_ref[...]
    xc = jnp.concatenate([xd, xt], axis=1)
    h1 = jnp.maximum(jnp.dot(xc, w1_ref[...], preferred_element_type=jnp.float32
                             ) + b1_ref[...], 0.0)
    h2 = jnp.maximum(jnp.dot(h1, w2_ref[...], preferred_element_type=jnp.float32
                             ) + b2_ref[...], 0.0)
    o_ref[...] = jnp.dot(h2, wo_ref[...], preferred_element_type=jnp.float32
                         ) + bo_ref[...]


def _head(pool, st4, w0, b0, tgt, embp, wf, wbig, bxt, w1, b1, w2, b2, wo, bo):
    grid = NGRAPH // BH
    cm = lambda i: (0, 0)
    return pl.pallas_call(
        _head_body,
        grid=(grid,),
        in_specs=[
            pl.BlockSpec((BH, 2 * FEAT), lambda i: (i, 0)),
            pl.BlockSpec((8, FEAT), cm),
            pl.BlockSpec((FEAT, 128), cm),
            pl.BlockSpec((1, 128), cm),
            pl.BlockSpec((BH, 1000), lambda i: (i, 0)),
            pl.BlockSpec((32, 128), cm),
            pl.BlockSpec((1000, 256), cm),
            pl.BlockSpec((128 * 256, 128), cm),
            pl.BlockSpec((1, 128), cm),
            pl.BlockSpec((256, 1024), cm),
            pl.BlockSpec((1, 1024), cm),
            pl.BlockSpec((1024, 256), cm),
            pl.BlockSpec((1, 256), cm),
            pl.BlockSpec((256, 128), cm),
            pl.BlockSpec((1, 128), cm),
        ],
        out_specs=pl.BlockSpec((BH, 128), lambda i: (i, 0)),
        out_shape=jax.ShapeDtypeStruct((NGRAPH, 128), jnp.float32),
    )(pool, st4, w0, b0, tgt, embp, wf, wbig, bxt, w1, b1, w2, b2, wo, bo)


def kernel(x, edge_index, batch, target, params):
    n = x.shape[0]
    e = edge_index.shape[1]
    src = edge_index[0]
    dst = edge_index[1]

    cpt = -(-e // (NTILE * CHUNK))
    cpt = -(-cpt // SB) * SB
    epad = cpt * NTILE * CHUNK
    src2d = jnp.concatenate(
        [src, jnp.zeros((epad - e,), jnp.int32)]).reshape(-1, CHUNK)
    dst2d = jnp.concatenate(
        [dst, jnp.full((epad - e,), n, jnp.int32)]).reshape(-1, CHUNK)
    zeros_pad = jnp.zeros((NPAD, FEAT), jnp.float32)

    cpt0 = -(-e // (NSUB * CHUNK0))
    cpt0 = -(-cpt0 // SB0) * SB0
    epad0 = cpt0 * NSUB * CHUNK0
    src2d0 = jnp.concatenate(
        [src, jnp.full((epad0 - e,), n, jnp.int32)]).reshape(-1, CHUNK0)
    dst2d0 = jnp.concatenate(
        [dst, jnp.zeros((epad0 - e,), jnp.int32)]).reshape(-1, CHUNK0)
    zeros_half = jnp.zeros((NPAD, HF), jnp.float32)

    x80 = jnp.concatenate(
        [x, jnp.ones((n, 1), jnp.float32), jnp.zeros((n, 1), jnp.float32)],
        axis=1)
    x80p = jnp.concatenate(
        [x80, jnp.zeros((NPAD - n, 2 * HF), jnp.float32)], axis=0)
    x3d = jnp.stack([x80p[:, :HF], x80p[:, HF:]])
    agg0 = _sc_aggregate0(x3d, src2d0, dst2d0, zeros_half)
    p0_0 = agg0[0, :n]
    p1_0 = agg0[1, :n]
    deg = p1_0[:, 38:39]

    w1_0 = params['gin0_l1']['W']
    wa = w1_0[:HF]
    wb = jnp.concatenate([w1_0[HF:], jnp.zeros((2, FEAT), jnp.float32)])
    z, stats = _gin0(x80p[:n, :HF], x80p[:n, HF:], p0_0, p1_0, wa, wb,
                     params['gin0_l1']['b'][None, :],
                     params['gin0_l2']['W'], params['gin0_l2']['b'][None, :])
    mu = stats[0] / n
    var = stats[1] / n - mu * mu
    s_bn = params['bn0_g'] / jnp.sqrt(var + BEPS)
    t_bn = params['bn0_b'] - mu * s_bn

    for i in range(1, 5):
        l1 = params['gin%d_l1' % i]
        l2 = params['gin%d_l2' % i]
        p = _sc_aggregate(z, src2d, dst2d, zeros_pad)
        z, stats = _gin(z, p[0, :n], p[1, :n], deg,
                        s_bn[None, :], t_bn[None, :],
                        l1['W'], l1['b'][None, :],
                        l2['W'], l2['b'][None, :])
        mu = stats[0] / n
        var = stats[1] / n - mu * mu
        s_bn = params['bn%d_g' % i] / jnp.sqrt(var + BEPS)
        t_bn = params['bn%d_b' % i] - mu * s_bn

    batch3d = batch.reshape(n // BN, 1, BN)
    pool = _pool(z, batch3d)

    st4 = jnp.concatenate([s_bn[None], t_bn[None],
                           jnp.zeros((6, FEAT), jnp.float32)], axis=0)

    def _rnd(a):
        return a.astype(jnp.bfloat16).astype(jnp.float32)

    embp = jnp.concatenate(
        [_rnd(params['emb']), jnp.zeros((6, 128), jnp.float32)], axis=0)
    wf = _rnd(params['conv_W']).transpose(1, 0, 2).reshape(1000, 256)
    wxtr = _rnd(params['fc1_xt']['W'])
    bias_rep = jnp.repeat(params['conv_b'], 121)
    bxt = (params['fc1_xt']['b'] + bias_rep @ wxtr)[None, :]
    wxt3 = wxtr.reshape(32, 121, 128)
    pp = jnp.arange(128)[:, None] - jnp.arange(8)[None, :]
    valid = (pp >= 0) & (pp <= 120)
    gathered = wxt3[:, jnp.clip(pp, 0, 120), :]
    wbig = jnp.where(valid[None, :, :, None], gathered,
                     0.0).transpose(1, 0, 2, 3).reshape(128 * 256, 128)
    wo = jnp.concatenate(
        [params['out']['W'], jnp.zeros((256, 127), jnp.float32)], axis=1)
    bo = jnp.concatenate(
        [params['out']['b'], jnp.zeros((127,), jnp.float32)])[None, :]

    res = _head(pool, st4,
                params['fc1_xd']['W'], params['fc1_xd']['b'][None, :],
                target, embp, wf,
                wbig, bxt,
                params['fc1']['W'], params['fc1']['b'][None, :],
                params['fc2']['W'], params['fc2']['b'][None, :],
                wo, bo)
    return res[:, :1]
